# Initial kernel scaffold; baseline (speedup 1.0000x reference)
import jax
import jax.numpy as jnp
from jax.experimental import pallas as pl


def kernel(xyz1, xyz2, points1, points2, W1, b1, W2, b2, W3, b3):
    raise NotImplementedError("write your pallas kernel here")


# speedup vs baseline: 25.1946x; 25.1946x over previous
import functools

import jax
import jax.numpy as jnp
from jax.experimental import pallas as pl


def _fused_kernel(x1_ref, x2_ref, p1_ref, p2_ref, w1_ref, b1_ref, w2_ref,
                  b2_ref, w3_ref, b3_ref, out_ref, *, bn, S):
    x1 = x1_ref[0]
    x2 = x2_ref[0]

    dot = jax.lax.dot_general(
        x1.astype(jnp.bfloat16), x2.astype(jnp.bfloat16),
        (((1,), (0,)), ((), ())),
        preferred_element_type=jnp.float32)
    x1sq = jnp.sum(x1 * x1, axis=1, keepdims=True)
    x2sq = jnp.sum(x2 * x2, axis=0, keepdims=True)
    dists = (-2.0 * dot + x1sq) + x2sq

    col = jax.lax.broadcasted_iota(jnp.int32, (bn, S), 1)
    big = jnp.float32(3.0e38)
    d = dists
    wmat = jnp.zeros((bn, S), dtype=jnp.float32)
    recips = []
    onehots = []
    for _ in range(3):
        m = jnp.min(d, axis=1, keepdims=True)
        amin = jnp.min(jnp.where(d == m, col, S), axis=1, keepdims=True)
        hit = col == amin
        recips.append(1.0 / (m + 1e-8))
        onehots.append(hit)
        d = jnp.where(hit, big, d)
    norm = recips[0] + recips[1] + recips[2]
    for r, hit in zip(recips, onehots):
        wmat = wmat + hit.astype(jnp.float32) * (r / norm)

    interp_t = jax.lax.dot_general(
        p2_ref[0], wmat, (((1,), (1,)), ((), ())),
        preferred_element_type=jnp.float32)

    new_t = jnp.concatenate([p1_ref[0], interp_t], axis=0)

    def layer(w_ref, b_ref, x):
        y = jax.lax.dot_general(
            w_ref[...].astype(jnp.bfloat16), x.astype(jnp.bfloat16),
            (((1,), (0,)), ((), ())),
            preferred_element_type=jnp.float32)
        return jax.nn.relu(y + b_ref[...])

    h = layer(w1_ref, b1_ref, new_t)
    h = layer(w2_ref, b2_ref, h)
    out_ref[0] = layer(w3_ref, b3_ref, h)


def kernel(xyz1, xyz2, points1, points2, W1, b1, W2, b2, W3, b3):
    B, C, N = xyz1.shape
    S = xyz2.shape[2]
    D = points1.shape[1]
    O = W3.shape[0]
    bn = 512
    grid = (B, N // bn)

    x1t = jnp.transpose(xyz1, (0, 2, 1))
    x1t = jnp.concatenate(
        [x1t, jnp.zeros((B, N, 8 - C), dtype=xyz1.dtype)], axis=2)
    x2p = jnp.concatenate(
        [xyz2, jnp.zeros((B, 8 - C, S), dtype=xyz2.dtype)], axis=1)

    b1c = b1.reshape(-1, 1)
    b2c = b2.reshape(-1, 1)
    b3c = b3.reshape(-1, 1)

    f = functools.partial(_fused_kernel, bn=bn, S=S)
    return pl.pallas_call(
        f,
        grid=grid,
        in_specs=[
            pl.BlockSpec((1, bn, 8), lambda b, n: (b, n, 0)),
            pl.BlockSpec((1, 8, S), lambda b, n: (b, 0, 0)),
            pl.BlockSpec((1, D, bn), lambda b, n: (b, 0, n)),
            pl.BlockSpec((1, D, S), lambda b, n: (b, 0, 0)),
            pl.BlockSpec(W1.shape, lambda b, n: (0, 0)),
            pl.BlockSpec(b1c.shape, lambda b, n: (0, 0)),
            pl.BlockSpec(W2.shape, lambda b, n: (0, 0)),
            pl.BlockSpec(b2c.shape, lambda b, n: (0, 0)),
            pl.BlockSpec(W3.shape, lambda b, n: (0, 0)),
            pl.BlockSpec(b3c.shape, lambda b, n: (0, 0)),
        ],
        out_specs=pl.BlockSpec((1, O, bn), lambda b, n: (b, 0, n)),
        out_shape=jax.ShapeDtypeStruct((B, O, N), jnp.float32),
    )(x1t, x2p, points1, points2, W1, b1c, W2, b2c, W3, b3c)


# speedup vs baseline: 26.9985x; 1.0716x over previous
import functools

import jax
import jax.numpy as jnp
from jax.experimental import pallas as pl


def _fused_kernel(x1_ref, x2_ref, p1_ref, p2_ref, w1a_ref, w1b_ref, b1_ref,
                  w2_ref, b2_ref, w3_ref, b3_ref, out_ref, *, bn, S):
    x1 = x1_ref[0]
    x2 = x2_ref[0]

    dot2 = jax.lax.dot_general(
        (-2.0 * x1).astype(jnp.bfloat16), x2.astype(jnp.bfloat16),
        (((1,), (0,)), ((), ())),
        preferred_element_type=jnp.float32)
    x1sq = jnp.sum(x1 * x1, axis=1, keepdims=True)
    x2sq = ((x2[0:1] * x2[0:1] + x2[1:2] * x2[1:2])
            + x2[2:3] * x2[2:3])
    dists = (dot2 + x1sq) + x2sq

    col = jax.lax.broadcasted_iota(jnp.int32, (bn, S), 1).astype(jnp.float32)
    big = jnp.float32(3.0e38)
    s_f = jnp.float32(S)
    d = dists
    ms = []
    hits = []
    for k in range(3):
        m = jnp.min(d, axis=1, keepdims=True)
        amin = jnp.min(jnp.where(d == m, col, s_f), axis=1, keepdims=True)
        hit = col == amin
        ms.append(m)
        hits.append(hit)
        if k < 2:
            d = jnp.where(hit, big, d)
    r0 = 1.0 / (ms[0] + 1e-8)
    r1 = 1.0 / (ms[1] + 1e-8)
    r2 = 1.0 / (ms[2] + 1e-8)
    norm = r0 + r1 + r2
    wmat = jnp.where(hits[0], r0 / norm, 0.0)
    wmat = jnp.where(hits[1], r1 / norm, wmat)
    wmat = jnp.where(hits[2], r2 / norm, wmat)

    interp_t = jax.lax.dot_general(
        p2_ref[0], wmat, (((1,), (1,)), ((), ())),
        preferred_element_type=jnp.float32)

    y = jax.lax.dot_general(
        w1a_ref[...], p1_ref[0], (((1,), (0,)), ((), ())),
        preferred_element_type=jnp.float32)
    y = y + jax.lax.dot_general(
        w1b_ref[...], interp_t.astype(jnp.bfloat16), (((1,), (0,)), ((), ())),
        preferred_element_type=jnp.float32)
    h = jax.nn.relu(y + b1_ref[...])

    h = jax.nn.relu(
        jax.lax.dot_general(w2_ref[...], h.astype(jnp.bfloat16),
                            (((1,), (0,)), ((), ())),
                            preferred_element_type=jnp.float32)
        + b2_ref[...])
    out_ref[0] = jax.nn.relu(
        jax.lax.dot_general(w3_ref[...], h.astype(jnp.bfloat16),
                            (((1,), (0,)), ((), ())),
                            preferred_element_type=jnp.float32)
        + b3_ref[...])


def kernel(xyz1, xyz2, points1, points2, W1, b1, W2, b2, W3, b3):
    B, C, N = xyz1.shape
    S = xyz2.shape[2]
    D = points1.shape[1]
    O = W3.shape[0]
    bn = 512
    grid = (B, N // bn)

    x1t = jnp.transpose(xyz1, (0, 2, 1))
    x1t = jnp.concatenate(
        [x1t, jnp.zeros((B, N, 8 - C), dtype=xyz1.dtype)], axis=2)
    x2p = jnp.concatenate(
        [xyz2, jnp.zeros((B, 8 - C, S), dtype=xyz2.dtype)], axis=1)

    p1b = points1.astype(jnp.bfloat16)
    w1a = W1[:, :D].astype(jnp.bfloat16)
    w1b = W1[:, D:].astype(jnp.bfloat16)
    w2b = W2.astype(jnp.bfloat16)
    w3b = W3.astype(jnp.bfloat16)
    b1c = b1.reshape(-1, 1)
    b2c = b2.reshape(-1, 1)
    b3c = b3.reshape(-1, 1)

    f = functools.partial(_fused_kernel, bn=bn, S=S)
    return pl.pallas_call(
        f,
        grid=grid,
        in_specs=[
            pl.BlockSpec((1, bn, 8), lambda b, n: (b, n, 0)),
            pl.BlockSpec((1, 8, S), lambda b, n: (b, 0, 0)),
            pl.BlockSpec((1, D, bn), lambda b, n: (b, 0, n)),
            pl.BlockSpec((1, D, S), lambda b, n: (b, 0, 0)),
            pl.BlockSpec(w1a.shape, lambda b, n: (0, 0)),
            pl.BlockSpec(w1b.shape, lambda b, n: (0, 0)),
            pl.BlockSpec(b1c.shape, lambda b, n: (0, 0)),
            pl.BlockSpec(w2b.shape, lambda b, n: (0, 0)),
            pl.BlockSpec(b2c.shape, lambda b, n: (0, 0)),
            pl.BlockSpec(w3b.shape, lambda b, n: (0, 0)),
            pl.BlockSpec(b3c.shape, lambda b, n: (0, 0)),
        ],
        out_specs=pl.BlockSpec((1, O, bn), lambda b, n: (b, 0, n)),
        out_shape=jax.ShapeDtypeStruct((B, O, N), jnp.float32),
    )(x1t, x2p, p1b, points2, w1a, w1b, b1c, w2b, b2c, w3b, b3c)
